# baseline (speedup 1.0000x reference)
import functools

import jax
import jax.numpy as jnp
from jax import lax
from jax.experimental import pallas as pl
from jax.experimental.pallas import tpu as pltpu
from jax.experimental.pallas import tpu_sc as plsc

_EMBED = 256
_HEADS = 8
_LEVELS = 4
_POINTS = 4
_B = 4
_NQ = 900
_NV = 5440
_D = _EMBED // _HEADS

_NDEST = _B * _NQ * _HEADS
_NW = 32
_DPW = _NDEST // _NW
_G = 6
_NCH = _DPW // _G
_ROWS = _G * 64
_IDXW = 128
_NDMA = _ROWS // _IDXW


def _prep_body(q_ref, v_ref, rpx_ref, rpy_ref, wv_ref, bv_ref, wox_ref,
               box_ref, woy_ref, boy_ref, wa_ref, ba_ref,
               tab_ref, xs_ref, ys_ref, at_ref):
    q = q_ref[0]
    tab_ref[0] = (jnp.dot(v_ref[0], wv_ref[...],
                          preferred_element_type=jnp.float32)
                  + bv_ref[...]).astype(jnp.bfloat16)
    offx = jnp.dot(q, wox_ref[...], preferred_element_type=jnp.float32) + box_ref[...]
    offy = jnp.dot(q, woy_ref[...], preferred_element_type=jnp.float32) + boy_ref[...]
    alog = jnp.dot(q, wa_ref[...], preferred_element_type=jnp.float32) + ba_ref[...]
    amax = jnp.max(alog, axis=1, keepdims=True)
    e = jnp.exp(alog - amax)
    gi = lax.broadcasted_iota(jnp.int32, (128, 128), 0) // 16
    gj = lax.broadcasted_iota(jnp.int32, (128, 128), 1) // 16
    gmat = (gi == gj).astype(jnp.float32)
    denom = jnp.dot(e, gmat, preferred_element_type=jnp.float32)
    at_ref[0] = e / denom
    lane = lax.broadcasted_iota(jnp.int32, (4, 128), 1)
    l_of = (lane // 4) % 4
    lev = lax.broadcasted_iota(jnp.int32, (4, 128), 0)
    sel = (lev == l_of).astype(jnp.float32)
    rpx = jnp.dot(rpx_ref[0], sel, preferred_element_type=jnp.float32)
    rpy = jnp.dot(rpy_ref[0], sel, preferred_element_type=jnp.float32)
    lane1 = lax.broadcasted_iota(jnp.int32, (1, 128), 1)
    wl = (64 >> ((lane1 // 4) % 4)).astype(jnp.float32)
    xs_ref[0] = rpx * wl + offx - 0.5
    ys_ref[0] = rpy * wl + offy - 0.5


def _prep_call(query, value, rpx, rpy, Wv, bv, Wox, box, Woy, boy, Wa, ba):
    f32 = jnp.float32
    return pl.pallas_call(
        _prep_body,
        grid=(_B,),
        in_specs=[
            pl.BlockSpec((1, _NQ, _EMBED), lambda b: (b, 0, 0)),
            pl.BlockSpec((1, _NV, _EMBED), lambda b: (b, 0, 0)),
            pl.BlockSpec((1, _NQ, _LEVELS), lambda b: (b, 0, 0)),
            pl.BlockSpec((1, _NQ, _LEVELS), lambda b: (b, 0, 0)),
            pl.BlockSpec((_EMBED, _EMBED), lambda b: (0, 0)),
            pl.BlockSpec((1, _EMBED), lambda b: (0, 0)),
            pl.BlockSpec((_EMBED, 128), lambda b: (0, 0)),
            pl.BlockSpec((1, 128), lambda b: (0, 0)),
            pl.BlockSpec((_EMBED, 128), lambda b: (0, 0)),
            pl.BlockSpec((1, 128), lambda b: (0, 0)),
            pl.BlockSpec((_EMBED, 128), lambda b: (0, 0)),
            pl.BlockSpec((1, 128), lambda b: (0, 0)),
        ],
        out_specs=[
            pl.BlockSpec((1, _NV, _EMBED), lambda b: (b, 0, 0)),
            pl.BlockSpec((1, _NQ, 128), lambda b: (b, 0, 0)),
            pl.BlockSpec((1, _NQ, 128), lambda b: (b, 0, 0)),
            pl.BlockSpec((1, _NQ, 128), lambda b: (b, 0, 0)),
        ],
        out_shape=[
            jax.ShapeDtypeStruct((_B, _NV, _EMBED), jnp.bfloat16),
            jax.ShapeDtypeStruct((_B, _NQ, 128), f32),
            jax.ShapeDtypeStruct((_B, _NQ, 128), f32),
            jax.ShapeDtypeStruct((_B, _NQ, 128), f32),
        ],
    )(query, value, rpx, rpy, Wv, bv, Wox, box, Woy, boy, Wa, ba)


def _samp_body(xs_hbm, ys_hbm, at_hbm, tab_hbm, out_hbm,
               xs_v, ys_v, a_v, idx_v, wt_v, rows_v, out_v,
               sem_in, sem_g0, sem_g1):
    wid = lax.axis_index("s") * 2 + lax.axis_index("c")
    base = wid * _DPW
    sems = (sem_g0, sem_g1)

    c1 = pltpu.async_copy(xs_hbm.at[pl.ds(base * 16, _DPW * 16)], xs_v, sem_in)
    c2 = pltpu.async_copy(ys_hbm.at[pl.ds(base * 16, _DPW * 16)], ys_v, sem_in)
    c3 = pltpu.async_copy(at_hbm.at[pl.ds(base * 16, _DPW * 16)], a_v, sem_in)
    c1.wait()
    c2.wait()
    c3.wait()

    def ci(v):
        return jnp.full((16,), v, jnp.int32)

    def cf(v):
        return jnp.full((16,), v, jnp.float32)

    def compute_and_fire(cix, p):
        lane = lax.iota(jnp.int32, 16)
        lev = lane >> ci(2)
        wlf = (ci(64) >> lev).astype(jnp.float32)
        wlm1f = wlf - cf(1.0)
        lstartf = (cf(16384.0) - wlf * wlf * cf(4.0)) / cf(3.0)
        for g in range(_G):
            d = base + cix * _G + g
            off = (cix * _G + g) * 16
            x = xs_v[pl.ds(off, 16)]
            y = ys_v[pl.ds(off, 16)]
            a = a_v[pl.ds(off, 16)]
            b_ix = d // (_NQ * _HEADS)
            m_ix = lax.rem(d, _HEADS)
            rbf = jnp.full((16,), (b_ix * (_NV * _HEADS) + m_ix)
                           .astype(jnp.float32), jnp.float32)
            xtf = x.astype(jnp.int32).astype(jnp.float32)
            x0f = xtf - jnp.maximum(jnp.sign(xtf - x), cf(0.0))
            ytf = y.astype(jnp.int32).astype(jnp.float32)
            y0f = ytf - jnp.maximum(jnp.sign(ytf - y), cf(0.0))
            fx = x - x0f
            fy = y - y0f
            x1f = x0f + cf(1.0)
            y1f = y0f + cf(1.0)

            def clp(v):
                return jnp.minimum(jnp.maximum(v, cf(0.0)), wlm1f)

            xc0, xc1 = clp(x0f), clp(x1f)
            yc0, yc1 = clp(y0f), clp(y1f)
            zf = cf(0.0)
            wx0 = jnp.where(x0f == xc0, cf(1.0) - fx, zf) * a
            wx1 = jnp.where(x1f == xc1, fx, zf) * a
            wy0 = jnp.where(y0f == yc0, cf(1.0) - fy, zf)
            wy1 = jnp.where(y1f == yc1, fy, zf)
            corners = (
                (xc0, yc0, wx0 * wy0),
                (xc1, yc0, wx1 * wy0),
                (xc0, yc1, wx0 * wy1),
                (xc1, yc1, wx1 * wy1),
            )
            for c, (xcc, ycc, wgt) in enumerate(corners):
                pos = g * 64 + c * 16
                idxf = (lstartf + ycc * wlf + xcc) * cf(float(_HEADS)) + rbf
                idx_v[p, pos // _IDXW, pl.ds(pos % _IDXW, 16)] = idxf.astype(jnp.int32)
                wt_v[p, pl.ds(pos, 16)] = wgt
        for k in range(_NDMA):
            pltpu.async_copy(tab_hbm.at[idx_v.at[p, k]],
                             rows_v.at[p, pl.ds(k * _IDXW, _IDXW)], sems[p])

    def wait_gathers(p):
        for k in range(_NDMA):
            pltpu.make_async_copy(tab_hbm.at[pl.ds(0, _IDXW)],
                                  rows_v.at[p, pl.ds(k * _IDXW, _IDXW)],
                                  sems[p]).wait()

    def accumulate(cix, p):
        for g in range(_G):
            acc = [jnp.zeros((16,), jnp.float32) for _ in range(8)]
            for jg in range(4):
                wv = wt_v[p, pl.ds(g * 64 + jg * 16, 16)]
                for j in range(16):
                    r = g * 64 + jg * 16 + j
                    wb = wv[jnp.full((16,), j, jnp.int32)]
                    u = rows_v[p, r, :]
                    ev = lax.bitcast_convert_type(u << ci(16), jnp.float32)
                    od = lax.bitcast_convert_type(u & ci(-65536), jnp.float32)
                    acc[jg * 2] = acc[jg * 2] + wb * ev
                    acc[jg * 2 + 1] = acc[jg * 2 + 1] + wb * od
            off = (cix * _G + g) * 32
            out_v[pl.ds(off, 16)] = acc[0] + acc[2] + acc[4] + acc[6]
            out_v[pl.ds(off + 16, 16)] = acc[1] + acc[3] + acc[5] + acc[7]

    compute_and_fire(0, 0)
    compute_and_fire(1, 1)

    def chunk2(i2, carry):
        for p in range(2):
            k = i2 * 2 + p
            wait_gathers(p)
            accumulate(k, p)

            @pl.when(k + 2 < _NCH)
            def _():
                compute_and_fire(k + 2, p)
        return carry

    lax.fori_loop(0, _NCH // 2, chunk2, 0)
    pltpu.sync_copy(out_v, out_hbm.at[pl.ds(base * 32, _DPW * 32)])


def _samp_call(xs, ys, at, tab):
    f32 = jnp.float32
    mesh = plsc.VectorSubcoreMesh(core_axis_name="c", subcore_axis_name="s")
    kern = pl.kernel(
        _samp_body,
        out_type=jax.ShapeDtypeStruct((_NDEST * _D,), f32),
        compiler_params=pltpu.CompilerParams(use_tc_tiling_on_sc=False),
        mesh=mesh,
        scratch_types=[
            pltpu.VMEM((_DPW * 16,), f32),
            pltpu.VMEM((_DPW * 16,), f32),
            pltpu.VMEM((_DPW * 16,), f32),
            pltpu.VMEM((2, _NDMA, _IDXW), jnp.int32),
            pltpu.VMEM((2, _ROWS), f32),
            pltpu.VMEM((2, _ROWS, _D // 2), jnp.int32),
            pltpu.VMEM((_DPW * 32,), f32),
            pltpu.SemaphoreType.DMA,
            pltpu.SemaphoreType.DMA,
            pltpu.SemaphoreType.DMA,
        ],
    )
    return kern(xs, ys, at, tab)


def _outproj_body(s_ref, q_ref, wout_ref, bout_ref, o_ref):
    o_ref[0] = (jnp.dot(s_ref[0], wout_ref[...],
                        preferred_element_type=jnp.float32)
                + bout_ref[...] + q_ref[0])


def _outproj_call(sampled, query, Wout, bout):
    return pl.pallas_call(
        _outproj_body,
        grid=(_B,),
        in_specs=[
            pl.BlockSpec((1, _NQ, _EMBED), lambda b: (b, 0, 0)),
            pl.BlockSpec((1, _NQ, _EMBED), lambda b: (b, 0, 0)),
            pl.BlockSpec((_EMBED, _EMBED), lambda b: (0, 0)),
            pl.BlockSpec((1, _EMBED), lambda b: (0, 0)),
        ],
        out_specs=pl.BlockSpec((1, _NQ, _EMBED), lambda b: (b, 0, 0)),
        out_shape=jax.ShapeDtypeStruct((_B, _NQ, _EMBED), jnp.float32),
    )(sampled, query, Wout, bout)


def kernel(query, value, reference_points, spatial_shapes, level_start_index,
           Wv, bv, Wo, bo, Wa, ba, Wout, bout):
    del spatial_shapes, level_start_index
    rp = reference_points.reshape(_B, _NQ, _LEVELS, 2)
    rpx = rp[..., 0]
    rpy = rp[..., 1]
    Wox = Wo[:, 0::2]
    Woy = Wo[:, 1::2]
    box = bo[0::2].reshape(1, 128)
    boy = bo[1::2].reshape(1, 128)
    tab, xs, ys, at = _prep_call(
        query, value, rpx, rpy, Wv, bv.reshape(1, -1), Wox, box, Woy, boy,
        Wa, ba.reshape(1, 128))
    tab_i32 = lax.bitcast_convert_type(
        tab.reshape(_B * _NV * _HEADS, _D // 2, 2), jnp.int32)
    sampled = _samp_call(
        xs.reshape(-1), ys.reshape(-1), at.reshape(-1), tab_i32)
    perm = [m * _D + c for m in range(_HEADS)
            for c in list(range(0, _D, 2)) + list(range(1, _D, 2))]
    Wout_p = Wout[jnp.array(perm, dtype=jnp.int32)]
    out = _outproj_call(sampled.reshape(_B, _NQ, _EMBED), query, Wout_p,
                        bout.reshape(1, -1))
    return out


# speedup vs baseline: 13.1368x; 13.1368x over previous
import functools

import jax
import jax.numpy as jnp
from jax import lax
from jax.experimental import pallas as pl
from jax.experimental.pallas import tpu as pltpu
from jax.experimental.pallas import tpu_sc as plsc

_EMBED = 256
_HEADS = 8
_LEVELS = 4
_POINTS = 4
_B = 4
_NQ = 900
_NV = 5440
_D = _EMBED // _HEADS

_NDEST = _B * _NQ * _HEADS
_NW = 32
_DPW = _NDEST // _NW
_G = 6
_NCH = _DPW // _G
_ROWS = _G * 64
_IDXW = 128
_NDMA = _ROWS // _IDXW


def _rne_bf16_bits(t):
    u = lax.bitcast_convert_type(t, jnp.int32)
    return ((u + 0x7FFF + ((u >> 16) & 1)) >> 16) & 0xFFFF


def _prep_body(q_ref, v_ref, rpx_ref, rpy_ref, wva_ref, bva_ref, wvb_ref,
               bvb_ref, wox_ref, box_ref, woy_ref, boy_ref, wa_ref, ba_ref,
               tab_ref, xs_ref, ys_ref, at_ref):
    q = q_ref[0]
    ta = (jnp.dot(v_ref[0], wva_ref[...], preferred_element_type=jnp.float32)
          + bva_ref[...])
    tb = (jnp.dot(v_ref[0], wvb_ref[...], preferred_element_type=jnp.float32)
          + bvb_ref[...])
    tab_ref[0] = _rne_bf16_bits(ta) | (_rne_bf16_bits(tb) << 16)
    offx = jnp.dot(q, wox_ref[...], preferred_element_type=jnp.float32) + box_ref[...]
    offy = jnp.dot(q, woy_ref[...], preferred_element_type=jnp.float32) + boy_ref[...]
    alog = jnp.dot(q, wa_ref[...], preferred_element_type=jnp.float32) + ba_ref[...]
    amax = jnp.max(alog, axis=1, keepdims=True)
    e = jnp.exp(alog - amax)
    gi = lax.broadcasted_iota(jnp.int32, (128, 128), 0) // 16
    gj = lax.broadcasted_iota(jnp.int32, (128, 128), 1) // 16
    gmat = (gi == gj).astype(jnp.float32)
    denom = jnp.dot(e, gmat, preferred_element_type=jnp.float32)
    at_ref[0] = e / denom
    lane = lax.broadcasted_iota(jnp.int32, (4, 128), 1)
    l_of = (lane // 4) % 4
    lev = lax.broadcasted_iota(jnp.int32, (4, 128), 0)
    sel = (lev == l_of).astype(jnp.float32)
    rpx = jnp.dot(rpx_ref[0], sel, preferred_element_type=jnp.float32)
    rpy = jnp.dot(rpy_ref[0], sel, preferred_element_type=jnp.float32)
    lane1 = lax.broadcasted_iota(jnp.int32, (1, 128), 1)
    wl = (64 >> ((lane1 // 4) % 4)).astype(jnp.float32)
    xs_ref[0] = rpx * wl + offx - 0.5
    ys_ref[0] = rpy * wl + offy - 0.5


def _prep_call(query, value, rpx, rpy, WvA, bvA, WvB, bvB,
               Wox, box, Woy, boy, Wa, ba):
    f32 = jnp.float32
    return pl.pallas_call(
        _prep_body,
        grid=(_B,),
        in_specs=[
            pl.BlockSpec((1, _NQ, _EMBED), lambda b: (b, 0, 0)),
            pl.BlockSpec((1, _NV, _EMBED), lambda b: (b, 0, 0)),
            pl.BlockSpec((1, _NQ, _LEVELS), lambda b: (b, 0, 0)),
            pl.BlockSpec((1, _NQ, _LEVELS), lambda b: (b, 0, 0)),
            pl.BlockSpec((_EMBED, 128), lambda b: (0, 0)),
            pl.BlockSpec((1, 128), lambda b: (0, 0)),
            pl.BlockSpec((_EMBED, 128), lambda b: (0, 0)),
            pl.BlockSpec((1, 128), lambda b: (0, 0)),
            pl.BlockSpec((_EMBED, 128), lambda b: (0, 0)),
            pl.BlockSpec((1, 128), lambda b: (0, 0)),
            pl.BlockSpec((_EMBED, 128), lambda b: (0, 0)),
            pl.BlockSpec((1, 128), lambda b: (0, 0)),
            pl.BlockSpec((_EMBED, 128), lambda b: (0, 0)),
            pl.BlockSpec((1, 128), lambda b: (0, 0)),
        ],
        out_specs=[
            pl.BlockSpec((1, _NV, 128), lambda b: (b, 0, 0)),
            pl.BlockSpec((1, _NQ, 128), lambda b: (b, 0, 0)),
            pl.BlockSpec((1, _NQ, 128), lambda b: (b, 0, 0)),
            pl.BlockSpec((1, _NQ, 128), lambda b: (b, 0, 0)),
        ],
        out_shape=[
            jax.ShapeDtypeStruct((_B, _NV, 128), jnp.int32),
            jax.ShapeDtypeStruct((_B, _NQ, 128), f32),
            jax.ShapeDtypeStruct((_B, _NQ, 128), f32),
            jax.ShapeDtypeStruct((_B, _NQ, 128), f32),
        ],
    )(query, value, rpx, rpy, WvA, bvA, WvB, bvB, Wox, box, Woy, boy, Wa, ba)


def _samp_body(xs_hbm, ys_hbm, at_hbm, tab_hbm, out_hbm,
               xs_v, ys_v, a_v, idx_v, wt_v, rows_v, out_v,
               sem_in, sem_g0, sem_g1):
    wid = lax.axis_index("s") * 2 + lax.axis_index("c")
    base = wid * _DPW
    sems = (sem_g0, sem_g1)

    c1 = pltpu.async_copy(xs_hbm.at[pl.ds(base * 16, _DPW * 16)], xs_v, sem_in)
    c2 = pltpu.async_copy(ys_hbm.at[pl.ds(base * 16, _DPW * 16)], ys_v, sem_in)
    c3 = pltpu.async_copy(at_hbm.at[pl.ds(base * 16, _DPW * 16)], a_v, sem_in)
    c1.wait()
    c2.wait()
    c3.wait()

    def ci(v):
        return jnp.full((16,), v, jnp.int32)

    def cf(v):
        return jnp.full((16,), v, jnp.float32)

    def compute_and_fire(cix, p):
        lane = lax.iota(jnp.int32, 16)
        lev = lane >> ci(2)
        wlf = (ci(64) >> lev).astype(jnp.float32)
        wlm1f = wlf - cf(1.0)
        lstartf = (cf(16384.0) - wlf * wlf * cf(4.0)) / cf(3.0)
        for g in range(_G):
            d = base + cix * _G + g
            off = (cix * _G + g) * 16
            x = xs_v[pl.ds(off, 16)]
            y = ys_v[pl.ds(off, 16)]
            a = a_v[pl.ds(off, 16)]
            b_ix = d // (_NQ * _HEADS)
            m_ix = lax.rem(d, _HEADS)
            rbf = jnp.full((16,), (b_ix * (_NV * _HEADS) + m_ix)
                           .astype(jnp.float32), jnp.float32)
            xtf = x.astype(jnp.int32).astype(jnp.float32)
            x0f = xtf - jnp.maximum(jnp.sign(xtf - x), cf(0.0))
            ytf = y.astype(jnp.int32).astype(jnp.float32)
            y0f = ytf - jnp.maximum(jnp.sign(ytf - y), cf(0.0))
            fx = x - x0f
            fy = y - y0f
            x1f = x0f + cf(1.0)
            y1f = y0f + cf(1.0)

            def clp(v):
                return jnp.minimum(jnp.maximum(v, cf(0.0)), wlm1f)

            xc0, xc1 = clp(x0f), clp(x1f)
            yc0, yc1 = clp(y0f), clp(y1f)
            zf = cf(0.0)
            wx0 = jnp.where(x0f == xc0, cf(1.0) - fx, zf) * a
            wx1 = jnp.where(x1f == xc1, fx, zf) * a
            wy0 = jnp.where(y0f == yc0, cf(1.0) - fy, zf)
            wy1 = jnp.where(y1f == yc1, fy, zf)
            corners = (
                (xc0, yc0, wx0 * wy0),
                (xc1, yc0, wx1 * wy0),
                (xc0, yc1, wx0 * wy1),
                (xc1, yc1, wx1 * wy1),
            )
            for c, (xcc, ycc, wgt) in enumerate(corners):
                pos = g * 64 + c * 16
                idxf = (lstartf + ycc * wlf + xcc) * cf(float(_HEADS)) + rbf
                idx_v[p, pos // _IDXW, pl.ds(pos % _IDXW, 16)] = idxf.astype(jnp.int32)
                wt_v[p, pl.ds(pos, 16)] = wgt
        for k in range(_NDMA):
            pltpu.async_copy(tab_hbm.at[idx_v.at[p, k]],
                             rows_v.at[p, pl.ds(k * _IDXW, _IDXW)], sems[p])

    def wait_gathers(p):
        for k in range(_NDMA):
            pltpu.make_async_copy(tab_hbm.at[pl.ds(0, _IDXW)],
                                  rows_v.at[p, pl.ds(k * _IDXW, _IDXW)],
                                  sems[p]).wait()

    def accumulate(cix, p):
        for g in range(_G):
            acc = [jnp.zeros((16,), jnp.float32) for _ in range(8)]
            for jg in range(4):
                wv = wt_v[p, pl.ds(g * 64 + jg * 16, 16)]
                for j in range(16):
                    r = g * 64 + jg * 16 + j
                    wb = wv[jnp.full((16,), j, jnp.int32)]
                    u = rows_v[p, r, :]
                    ev = lax.bitcast_convert_type(u << ci(16), jnp.float32)
                    od = lax.bitcast_convert_type(u & ci(-65536), jnp.float32)
                    acc[jg * 2] = acc[jg * 2] + wb * ev
                    acc[jg * 2 + 1] = acc[jg * 2 + 1] + wb * od
            off = (cix * _G + g) * 32
            out_v[pl.ds(off, 16)] = acc[0] + acc[2] + acc[4] + acc[6]
            out_v[pl.ds(off + 16, 16)] = acc[1] + acc[3] + acc[5] + acc[7]

    compute_and_fire(0, 0)
    compute_and_fire(1, 1)

    def chunk2(i2, carry):
        for p in range(2):
            k = i2 * 2 + p
            wait_gathers(p)
            accumulate(k, p)

            @pl.when(k + 2 < _NCH)
            def _():
                compute_and_fire(k + 2, p)
        return carry

    lax.fori_loop(0, _NCH // 2, chunk2, 0)
    pltpu.sync_copy(out_v, out_hbm.at[pl.ds(base * 32, _DPW * 32)])


def _samp_call(xs, ys, at, tab):
    f32 = jnp.float32
    mesh = plsc.VectorSubcoreMesh(core_axis_name="c", subcore_axis_name="s")
    kern = pl.kernel(
        _samp_body,
        out_type=jax.ShapeDtypeStruct((_NDEST * _D,), f32),
        compiler_params=pltpu.CompilerParams(use_tc_tiling_on_sc=False),
        mesh=mesh,
        scratch_types=[
            pltpu.VMEM((_DPW * 16,), f32),
            pltpu.VMEM((_DPW * 16,), f32),
            pltpu.VMEM((_DPW * 16,), f32),
            pltpu.VMEM((2, _NDMA, _IDXW), jnp.int32),
            pltpu.VMEM((2, _ROWS), f32),
            pltpu.VMEM((2, _ROWS, _D // 2), jnp.int32),
            pltpu.VMEM((_DPW * 32,), f32),
            pltpu.SemaphoreType.DMA,
            pltpu.SemaphoreType.DMA,
            pltpu.SemaphoreType.DMA,
        ],
    )
    return kern(xs, ys, at, tab)


def _outproj_body(s_ref, q_ref, wout_ref, bout_ref, o_ref):
    o_ref[0] = (jnp.dot(s_ref[0], wout_ref[...],
                        preferred_element_type=jnp.float32)
                + bout_ref[...] + q_ref[0])


def _outproj_call(sampled, query, Wout, bout):
    return pl.pallas_call(
        _outproj_body,
        grid=(_B,),
        in_specs=[
            pl.BlockSpec((1, _NQ, _EMBED), lambda b: (b, 0, 0)),
            pl.BlockSpec((1, _NQ, _EMBED), lambda b: (b, 0, 0)),
            pl.BlockSpec((_EMBED, _EMBED), lambda b: (0, 0)),
            pl.BlockSpec((1, _EMBED), lambda b: (0, 0)),
        ],
        out_specs=pl.BlockSpec((1, _NQ, _EMBED), lambda b: (b, 0, 0)),
        out_shape=jax.ShapeDtypeStruct((_B, _NQ, _EMBED), jnp.float32),
    )(sampled, query, Wout, bout)


def kernel(query, value, reference_points, spatial_shapes, level_start_index,
           Wv, bv, Wo, bo, Wa, ba, Wout, bout):
    del spatial_shapes, level_start_index
    rp = reference_points.reshape(_B, _NQ, _LEVELS, 2)
    rpx = rp[..., 0]
    rpy = rp[..., 1]
    Wox = Wo[:, 0::2]
    Woy = Wo[:, 1::2]
    box = bo[0::2].reshape(1, 128)
    boy = bo[1::2].reshape(1, 128)
    colsA = jnp.array([m * _D + k for m in range(_HEADS)
                       for k in range(_D // 2)], dtype=jnp.int32)
    colsB = colsA + (_D // 2)
    WvA = Wv[:, colsA]
    WvB = Wv[:, colsB]
    bvA = bv[colsA].reshape(1, 128)
    bvB = bv[colsB].reshape(1, 128)
    tab, xs, ys, at = _prep_call(
        query, value, rpx, rpy, WvA, bvA, WvB, bvB, Wox, box, Woy, boy,
        Wa, ba.reshape(1, 128))
    sampled = _samp_call(
        xs.reshape(-1), ys.reshape(-1), at.reshape(-1),
        tab.reshape(_B * _NV * _HEADS, _D // 2))
    out = _outproj_call(sampled.reshape(_B, _NQ, _EMBED), query, Wout,
                        bout.reshape(1, -1))
    return out


# speedup vs baseline: 13.3670x; 1.0175x over previous
import functools

import jax
import jax.numpy as jnp
from jax import lax
from jax.experimental import pallas as pl
from jax.experimental.pallas import tpu as pltpu
from jax.experimental.pallas import tpu_sc as plsc

_EMBED = 256
_HEADS = 8
_LEVELS = 4
_POINTS = 4
_B = 4
_NQ = 900
_NV = 5440
_D = _EMBED // _HEADS

_NDEST = _B * _NQ * _HEADS
_NW = 32
_DPW = _NDEST // _NW
_G = 12
_NCH = _DPW // _G
_ROWS = _G * 64
_IDXW = 128
_NDMA = _ROWS // _IDXW


def _rne_bf16_bits(t):
    u = lax.bitcast_convert_type(t, jnp.int32)
    return ((u + 0x7FFF + ((u >> 16) & 1)) >> 16) & 0xFFFF


def _prep_body(q_ref, v_ref, rpx_ref, rpy_ref, wva_ref, bva_ref, wvb_ref,
               bvb_ref, wox_ref, box_ref, woy_ref, boy_ref, wa_ref, ba_ref,
               tab_ref, xs_ref, ys_ref, at_ref):
    q = q_ref[0]
    ta = (jnp.dot(v_ref[0], wva_ref[...], preferred_element_type=jnp.float32)
          + bva_ref[...])
    tb = (jnp.dot(v_ref[0], wvb_ref[...], preferred_element_type=jnp.float32)
          + bvb_ref[...])
    tab_ref[0] = _rne_bf16_bits(ta) | (_rne_bf16_bits(tb) << 16)
    offx = jnp.dot(q, wox_ref[...], preferred_element_type=jnp.float32) + box_ref[...]
    offy = jnp.dot(q, woy_ref[...], preferred_element_type=jnp.float32) + boy_ref[...]
    alog = jnp.dot(q, wa_ref[...], preferred_element_type=jnp.float32) + ba_ref[...]
    amax = jnp.max(alog, axis=1, keepdims=True)
    e = jnp.exp(alog - amax)
    gi = lax.broadcasted_iota(jnp.int32, (128, 128), 0) // 16
    gj = lax.broadcasted_iota(jnp.int32, (128, 128), 1) // 16
    gmat = (gi == gj).astype(jnp.float32)
    denom = jnp.dot(e, gmat, preferred_element_type=jnp.float32)
    at_ref[0] = e / denom
    lane = lax.broadcasted_iota(jnp.int32, (4, 128), 1)
    l_of = (lane // 4) % 4
    lev = lax.broadcasted_iota(jnp.int32, (4, 128), 0)
    sel = (lev == l_of).astype(jnp.float32)
    rpx = jnp.dot(rpx_ref[0], sel, preferred_element_type=jnp.float32)
    rpy = jnp.dot(rpy_ref[0], sel, preferred_element_type=jnp.float32)
    lane1 = lax.broadcasted_iota(jnp.int32, (1, 128), 1)
    wl = (64 >> ((lane1 // 4) % 4)).astype(jnp.float32)
    xs_ref[0] = rpx * wl + offx - 0.5
    ys_ref[0] = rpy * wl + offy - 0.5


def _prep_call(query, value, rpx, rpy, WvA, bvA, WvB, bvB,
               Wox, box, Woy, boy, Wa, ba):
    f32 = jnp.float32
    return pl.pallas_call(
        _prep_body,
        grid=(_B,),
        in_specs=[
            pl.BlockSpec((1, _NQ, _EMBED), lambda b: (b, 0, 0)),
            pl.BlockSpec((1, _NV, _EMBED), lambda b: (b, 0, 0)),
            pl.BlockSpec((1, _NQ, _LEVELS), lambda b: (b, 0, 0)),
            pl.BlockSpec((1, _NQ, _LEVELS), lambda b: (b, 0, 0)),
            pl.BlockSpec((_EMBED, 128), lambda b: (0, 0)),
            pl.BlockSpec((1, 128), lambda b: (0, 0)),
            pl.BlockSpec((_EMBED, 128), lambda b: (0, 0)),
            pl.BlockSpec((1, 128), lambda b: (0, 0)),
            pl.BlockSpec((_EMBED, 128), lambda b: (0, 0)),
            pl.BlockSpec((1, 128), lambda b: (0, 0)),
            pl.BlockSpec((_EMBED, 128), lambda b: (0, 0)),
            pl.BlockSpec((1, 128), lambda b: (0, 0)),
            pl.BlockSpec((_EMBED, 128), lambda b: (0, 0)),
            pl.BlockSpec((1, 128), lambda b: (0, 0)),
        ],
        out_specs=[
            pl.BlockSpec((1, _NV, 128), lambda b: (b, 0, 0)),
            pl.BlockSpec((1, _NQ, 128), lambda b: (b, 0, 0)),
            pl.BlockSpec((1, _NQ, 128), lambda b: (b, 0, 0)),
            pl.BlockSpec((1, _NQ, 128), lambda b: (b, 0, 0)),
        ],
        out_shape=[
            jax.ShapeDtypeStruct((_B, _NV, 128), jnp.int32),
            jax.ShapeDtypeStruct((_B, _NQ, 128), f32),
            jax.ShapeDtypeStruct((_B, _NQ, 128), f32),
            jax.ShapeDtypeStruct((_B, _NQ, 128), f32),
        ],
    )(query, value, rpx, rpy, WvA, bvA, WvB, bvB, Wox, box, Woy, boy, Wa, ba)


def _samp_body(xs_hbm, ys_hbm, at_hbm, tab_hbm, out_hbm,
               xs_v, ys_v, a_v, idx_v, wt_v, rows_v, out_v,
               sem_in, sem_g0, sem_g1):
    wid = lax.axis_index("s") * 2 + lax.axis_index("c")
    base = wid * _DPW
    sems = (sem_g0, sem_g1)

    c1 = pltpu.async_copy(xs_hbm.at[pl.ds(base * 16, _DPW * 16)], xs_v, sem_in)
    c2 = pltpu.async_copy(ys_hbm.at[pl.ds(base * 16, _DPW * 16)], ys_v, sem_in)
    c3 = pltpu.async_copy(at_hbm.at[pl.ds(base * 16, _DPW * 16)], a_v, sem_in)
    c1.wait()
    c2.wait()
    c3.wait()

    def ci(v):
        return jnp.full((16,), v, jnp.int32)

    def cf(v):
        return jnp.full((16,), v, jnp.float32)

    def compute_and_fire(cix, p):
        lane = lax.iota(jnp.int32, 16)
        lev = lane >> ci(2)
        wlf = (ci(64) >> lev).astype(jnp.float32)
        wlm1f = wlf - cf(1.0)
        lstartf = (cf(16384.0) - wlf * wlf * cf(4.0)) / cf(3.0)
        for g in range(_G):
            d = base + cix * _G + g
            off = (cix * _G + g) * 16
            x = xs_v[pl.ds(off, 16)]
            y = ys_v[pl.ds(off, 16)]
            a = a_v[pl.ds(off, 16)]
            b_ix = d // (_NQ * _HEADS)
            m_ix = lax.rem(d, _HEADS)
            rbf = jnp.full((16,), (b_ix * (_NV * _HEADS) + m_ix)
                           .astype(jnp.float32), jnp.float32)
            xtf = x.astype(jnp.int32).astype(jnp.float32)
            x0f = xtf - jnp.maximum(jnp.sign(xtf - x), cf(0.0))
            ytf = y.astype(jnp.int32).astype(jnp.float32)
            y0f = ytf - jnp.maximum(jnp.sign(ytf - y), cf(0.0))
            fx = x - x0f
            fy = y - y0f
            x1f = x0f + cf(1.0)
            y1f = y0f + cf(1.0)

            def clp(v):
                return jnp.minimum(jnp.maximum(v, cf(0.0)), wlm1f)

            xc0, xc1 = clp(x0f), clp(x1f)
            yc0, yc1 = clp(y0f), clp(y1f)
            zf = cf(0.0)
            wx0 = jnp.where(x0f == xc0, cf(1.0) - fx, zf) * a
            wx1 = jnp.where(x1f == xc1, fx, zf) * a
            wy0 = jnp.where(y0f == yc0, cf(1.0) - fy, zf)
            wy1 = jnp.where(y1f == yc1, fy, zf)
            corners = (
                (xc0, yc0, wx0 * wy0),
                (xc1, yc0, wx1 * wy0),
                (xc0, yc1, wx0 * wy1),
                (xc1, yc1, wx1 * wy1),
            )
            for c, (xcc, ycc, wgt) in enumerate(corners):
                pos = g * 64 + c * 16
                idxf = (lstartf + ycc * wlf + xcc) * cf(float(_HEADS)) + rbf
                idx_v[p, pos // _IDXW, pl.ds(pos % _IDXW, 16)] = idxf.astype(jnp.int32)
                wt_v[p, pl.ds(pos, 16)] = wgt
        for k in range(_NDMA):
            pltpu.async_copy(tab_hbm.at[idx_v.at[p, k]],
                             rows_v.at[p, pl.ds(k * _IDXW, _IDXW)], sems[p])

    def wait_gathers(p):
        for k in range(_NDMA):
            pltpu.make_async_copy(tab_hbm.at[pl.ds(0, _IDXW)],
                                  rows_v.at[p, pl.ds(k * _IDXW, _IDXW)],
                                  sems[p]).wait()

    def accumulate(cix, p):
        for g in range(_G):
            acc = [jnp.zeros((16,), jnp.float32) for _ in range(8)]
            for jg in range(4):
                wv = wt_v[p, pl.ds(g * 64 + jg * 16, 16)]
                for j in range(16):
                    r = g * 64 + jg * 16 + j
                    wb = wv[jnp.full((16,), j, jnp.int32)]
                    u = rows_v[p, r, :]
                    ev = lax.bitcast_convert_type(u << ci(16), jnp.float32)
                    od = lax.bitcast_convert_type(u, jnp.float32)
                    acc[jg * 2] = acc[jg * 2] + wb * ev
                    acc[jg * 2 + 1] = acc[jg * 2 + 1] + wb * od
            off = (cix * _G + g) * 32
            out_v[pl.ds(off, 16)] = acc[0] + acc[2] + acc[4] + acc[6]
            out_v[pl.ds(off + 16, 16)] = acc[1] + acc[3] + acc[5] + acc[7]

    compute_and_fire(0, 0)
    compute_and_fire(1, 1)

    def chunk2(i2, carry):
        for p in range(2):
            k = i2 * 2 + p
            wait_gathers(p)
            accumulate(k, p)

            @pl.when(k + 2 < _NCH)
            def _():
                compute_and_fire(k + 2, p)
        return carry

    lax.fori_loop(0, _NCH // 2, chunk2, 0)
    pltpu.sync_copy(out_v, out_hbm.at[pl.ds(base * 32, _DPW * 32)])


def _samp_call(xs, ys, at, tab):
    f32 = jnp.float32
    mesh = plsc.VectorSubcoreMesh(core_axis_name="c", subcore_axis_name="s")
    kern = pl.kernel(
        _samp_body,
        out_type=jax.ShapeDtypeStruct((_NDEST * _D,), f32),
        compiler_params=pltpu.CompilerParams(use_tc_tiling_on_sc=False),
        mesh=mesh,
        scratch_types=[
            pltpu.VMEM((_DPW * 16,), f32),
            pltpu.VMEM((_DPW * 16,), f32),
            pltpu.VMEM((_DPW * 16,), f32),
            pltpu.VMEM((2, _NDMA, _IDXW), jnp.int32),
            pltpu.VMEM((2, _ROWS), f32),
            pltpu.VMEM((2, _ROWS, _D // 2), jnp.int32),
            pltpu.VMEM((_DPW * 32,), f32),
            pltpu.SemaphoreType.DMA,
            pltpu.SemaphoreType.DMA,
            pltpu.SemaphoreType.DMA,
        ],
    )
    return kern(xs, ys, at, tab)


def _outproj_body(s_ref, q_ref, wout_ref, bout_ref, o_ref):
    o_ref[0] = (jnp.dot(s_ref[0], wout_ref[...],
                        preferred_element_type=jnp.float32)
                + bout_ref[...] + q_ref[0])


def _outproj_call(sampled, query, Wout, bout):
    return pl.pallas_call(
        _outproj_body,
        grid=(_B,),
        in_specs=[
            pl.BlockSpec((1, _NQ, _EMBED), lambda b: (b, 0, 0)),
            pl.BlockSpec((1, _NQ, _EMBED), lambda b: (b, 0, 0)),
            pl.BlockSpec((_EMBED, _EMBED), lambda b: (0, 0)),
            pl.BlockSpec((1, _EMBED), lambda b: (0, 0)),
        ],
        out_specs=pl.BlockSpec((1, _NQ, _EMBED), lambda b: (b, 0, 0)),
        out_shape=jax.ShapeDtypeStruct((_B, _NQ, _EMBED), jnp.float32),
    )(sampled, query, Wout, bout)


def kernel(query, value, reference_points, spatial_shapes, level_start_index,
           Wv, bv, Wo, bo, Wa, ba, Wout, bout):
    del spatial_shapes, level_start_index
    rp = reference_points.reshape(_B, _NQ, _LEVELS, 2)
    rpx = rp[..., 0]
    rpy = rp[..., 1]
    Wox = Wo[:, 0::2]
    Woy = Wo[:, 1::2]
    box = bo[0::2].reshape(1, 128)
    boy = bo[1::2].reshape(1, 128)
    colsA = jnp.array([m * _D + k for m in range(_HEADS)
                       for k in range(_D // 2)], dtype=jnp.int32)
    colsB = colsA + (_D // 2)
    WvA = Wv[:, colsA]
    WvB = Wv[:, colsB]
    bvA = bv[colsA].reshape(1, 128)
    bvB = bv[colsB].reshape(1, 128)
    tab, xs, ys, at = _prep_call(
        query, value, rpx, rpy, WvA, bvA, WvB, bvB, Wox, box, Woy, boy,
        Wa, ba.reshape(1, 128))
    sampled = _samp_call(
        xs.reshape(-1), ys.reshape(-1), at.reshape(-1),
        tab.reshape(_B * _NV * _HEADS, _D // 2))
    out = _outproj_call(sampled.reshape(_B, _NQ, _EMBED), query, Wout,
                        bout.reshape(1, -1))
    return out
